# baseline (speedup 1.0000x reference)
import functools

import jax
import jax.numpy as jnp
from jax import lax
from jax.experimental import pallas as pl
from jax.experimental.pallas import tpu as pltpu

ROWS = 128
COLS = 32768
BR = 16
GRID = ROWS // BR


def _tc_body(x_ref, o_ref):
    x = x_ref[...]
    idx = jnp.argmin(x, axis=1).astype(jnp.int32)
    o_ref[...] = idx.reshape(1, 1, BR)


@functools.partial(jax.jit)
def kernel(x):
    out = pl.pallas_call(
        _tc_body,
        out_shape=jax.ShapeDtypeStruct((GRID, 1, BR), jnp.int32),
        grid=(GRID,),
        in_specs=[pl.BlockSpec((BR, COLS), lambda i: (i, 0))],
        out_specs=pl.BlockSpec((1, 1, BR), lambda i: (i, 0, 0)),
    )(x)
    return out.reshape(ROWS)


# speedup vs baseline: 1.1843x; 1.1843x over previous
import functools

import jax
import jax.numpy as jnp
from jax import lax
from jax.experimental import pallas as pl
from jax.experimental.pallas import tpu as pltpu

ROWS = 128
COLS = 32768
BR = 64
GRID = ROWS // BR


def _tc_body(x_ref, o_ref):
    x = x_ref[...]
    idx = jnp.argmin(x, axis=1).astype(jnp.int32)
    o_ref[...] = idx.reshape(1, 1, BR)


@functools.partial(jax.jit)
def kernel(x):
    out = pl.pallas_call(
        _tc_body,
        out_shape=jax.ShapeDtypeStruct((GRID, 1, BR), jnp.int32),
        grid=(GRID,),
        in_specs=[pl.BlockSpec((BR, COLS), lambda i: (i, 0))],
        out_specs=pl.BlockSpec((1, 1, BR), lambda i: (i, 0, 0)),
    )(x)
    return out.reshape(ROWS)
